# baseline (speedup 1.0000x reference)
import functools

import jax
import jax.numpy as jnp
import numpy as np
from jax import lax
from jax.experimental import pallas as pl
from jax.experimental.pallas import tpu as pltpu
from jax.experimental.pallas import tpu_sc as plsc

L_LEVELS = 16
T_SIZE = 2 ** 14
N_FEAT = 2
N_MIN = 16
N_MAX = 1024
_B_GROWTH = np.exp((np.log(N_MAX) - np.log(N_MIN)) / (L_LEVELS - 1))
_P1 = np.uint32(2654435761)

_NLS = [int(np.floor(N_MIN * (_B_GROWTH ** l))) for l in range(L_LEVELS)]

_NC = 2
_NS = 16
_NW = _NC * _NS
_VL = 16


def _sc_hash_encode(t0, t1, px, py):
    Q = px.shape[0]
    assert Q % (_NW * _VL) == 0
    chunk = Q // _NW
    n_grp = chunk // _VL
    unroll = 4
    assert n_grp % unroll == 0
    mesh = plsc.VectorSubcoreMesh(core_axis_name="c", subcore_axis_name="s")

    @functools.partial(
        pl.kernel,
        out_type=(
            jax.ShapeDtypeStruct((L_LEVELS, Q), jnp.float32),
            jax.ShapeDtypeStruct((L_LEVELS, Q), jnp.float32),
        ),
        mesh=mesh,
        compiler_params=pltpu.CompilerParams(needs_layout_passes=False),
        scratch_types=[
            pltpu.VMEM((chunk,), jnp.float32),
            pltpu.VMEM((chunk,), jnp.float32),
            pltpu.VMEM((T_SIZE,), jnp.float32),
            pltpu.VMEM((T_SIZE,), jnp.float32),
            pltpu.VMEM((T_SIZE,), jnp.float32),
            pltpu.VMEM((T_SIZE,), jnp.float32),
            pltpu.VMEM((chunk,), jnp.float32),
            pltpu.VMEM((chunk,), jnp.float32),
            pltpu.VMEM((chunk,), jnp.float32),
            pltpu.VMEM((chunk,), jnp.float32),
            pltpu.SemaphoreType.DMA,
            pltpu.SemaphoreType.DMA,
            pltpu.SemaphoreType.DMA,
            pltpu.SemaphoreType.DMA,
        ],
    )
    def enc(t0_hbm, t1_hbm, px_hbm, py_hbm, o0_hbm, o1_hbm,
            px_v, py_v, t0a, t0b, t1a, t1b, a0a, a0b, a1a, a1b,
            si0, si1, so0, so1):
        wid = lax.axis_index("s") * _NC + lax.axis_index("c")
        base = wid * chunk
        tb0 = [t0a, t0b]
        tb1 = [t1a, t1b]
        ab0 = [a0a, a0b]
        ab1 = [a1a, a1b]
        sin = [si0, si1]
        sout = [so0, so1]
        in_copies = {}
        out_copies = {}

        def start_in(l):
            p = l & 1
            c0 = pltpu.make_async_copy(t0_hbm.at[l], tb0[p], sin[p])
            c1 = pltpu.make_async_copy(t1_hbm.at[l], tb1[p], sin[p])
            c0.start()
            c1.start()
            in_copies[l] = (c0, c1)

        start_in(0)
        pltpu.sync_copy(px_hbm.at[pl.ds(base, chunk)], px_v)
        pltpu.sync_copy(py_hbm.at[pl.ds(base, chunk)], py_v)
        for l in range(L_LEVELS):
            p = l & 1
            nl = _NLS[l]
            dense = (nl + 1) ** 2 <= T_SIZE
            if l + 1 < L_LEVELS:
                start_in(l + 1)
            for c in in_copies.pop(l):
                c.wait()
            if l >= 2:
                for c in out_copies.pop(l - 2):
                    c.wait()
            tab0_v = tb0[p]
            tab1_v = tb1[p]
            a0_v = ab0[p]
            a1_v = ab1[p]

            def grp(g, _, nl=nl, dense=dense, tab0_v=tab0_v, tab1_v=tab1_v,
                    a0_v=a0_v, a1_v=a1_v):
                for u in range(unroll):
                    s = pl.ds(g * (_VL * unroll) + u * _VL, _VL)
                    x = px_v[s] * jnp.float32(nl)
                    y = py_v[s] * jnp.float32(nl)
                    ix = x.astype(jnp.int32)
                    iy = y.astype(jnp.int32)
                    fx = x - ix.astype(jnp.float32)
                    fy = y - iy.astype(jnp.float32)
                    if dense:
                        b = ix * (nl + 1) + iy
                        i00 = b
                        i01 = b + 1
                        i10 = b + (nl + 1)
                        i11 = b + (nl + 2)
                    else:
                        hx0 = ix.astype(jnp.uint32)
                        hx1 = hx0 + jnp.uint32(1)
                        hy0 = iy.astype(jnp.uint32) * _P1
                        hy1 = hy0 + _P1
                        m = jnp.uint32(T_SIZE - 1)
                        i00 = ((hx0 ^ hy0) & m).astype(jnp.int32)
                        i01 = ((hx0 ^ hy1) & m).astype(jnp.int32)
                        i10 = ((hx1 ^ hy0) & m).astype(jnp.int32)
                        i11 = ((hx1 ^ hy1) & m).astype(jnp.int32)
                    gx0 = 1.0 - fx
                    gy0 = 1.0 - fy
                    w00 = gx0 * gy0
                    w01 = gx0 * fy
                    w10 = fx * gy0
                    w11 = fx * fy
                    a0 = (w00 * plsc.load_gather(tab0_v, [i00])
                          + w01 * plsc.load_gather(tab0_v, [i01])
                          + w10 * plsc.load_gather(tab0_v, [i10])
                          + w11 * plsc.load_gather(tab0_v, [i11]))
                    a1 = (w00 * plsc.load_gather(tab1_v, [i00])
                          + w01 * plsc.load_gather(tab1_v, [i01])
                          + w10 * plsc.load_gather(tab1_v, [i10])
                          + w11 * plsc.load_gather(tab1_v, [i11]))
                    a0_v[s] = a0
                    a1_v[s] = a1
                return 0

            lax.fori_loop(0, n_grp // unroll, grp, 0)
            oc0 = pltpu.make_async_copy(
                a0_v, o0_hbm.at[l, pl.ds(base, chunk)], sout[p])
            oc1 = pltpu.make_async_copy(
                a1_v, o1_hbm.at[l, pl.ds(base, chunk)], sout[p])
            oc0.start()
            oc1.start()
            out_copies[l] = (oc0, oc1)
        for ll in (L_LEVELS - 2, L_LEVELS - 1):
            for c in out_copies.pop(ll):
                c.wait()

    return enc(t0, t1, px, py)


_BB = 512


def _leaky(x):
    return jnp.where(x >= 0, x, jnp.float32(0.01) * x)


def _sel1():
    S = np.zeros((1024, 64, 24), np.float32)
    for a in range(4):
        for t in range(64):
            for k in range(3):
                for c in range(2):
                    ch = 2 * a + c
                    col = a * 256 + 4 * t + (2 * k + c if k < 2 else 4 + c)
                    if col < (a + 1) * 256:
                        S[col, t, k * 8 + ch] = 1.0
    return S.reshape(1024 * 64, 24)


def _sel(win, wout, cin):
    S = np.zeros((win * cin, wout, 3 * cin), np.float32)
    for t in range(wout):
        for k in range(3):
            w = 2 * t + k
            if w < win:
                for ch in range(cin):
                    S[w * cin + ch, t, k * cin + ch] = 1.0
    return S.reshape(win * cin * wout, 3 * cin)


_S1 = _sel1()
_S2 = _sel(64, 32, 16)
_S3 = _sel(32, 16, 16)


def _lidar_side(x, G1, b1, G2, b2, G3, b3, lns, lnb):
    bf = jnp.bfloat16
    f32 = jnp.float32
    y1 = _leaky(jnp.dot(x.astype(bf), G1, preferred_element_type=f32) + b1)
    y2 = _leaky(jnp.dot(y1.astype(bf), G2, preferred_element_type=f32) + b2)
    y3 = jnp.dot(y2.astype(bf), G3, preferred_element_type=f32) + b3
    mu = jnp.mean(y3, axis=-1, keepdims=True)
    var = jnp.mean((y3 - mu) ** 2, axis=-1, keepdims=True)
    return _leaky((y3 - mu) * lax.rsqrt(var + 1e-6) * lns + lnb)


def _lidar_body(xf_ref, xr_ref, *refs):
    fw = [r[...] for r in refs[0:8]]
    rw = [r[...] for r in refs[8:16]]
    o_ref = refs[16]
    o_ref[:, 0:256] = _lidar_side(xf_ref[...], *fw)
    o_ref[:, 256:512] = _lidar_side(xr_ref[...], *rw)


def _prep_w(W1, b1, W2, b2, W3, b3, ln_s, ln_b):
    bf = jnp.bfloat16
    G1 = jnp.dot(jnp.asarray(_S1), W1.reshape(24, 16)).reshape(1024, 1024).astype(bf)
    G2 = jnp.dot(jnp.asarray(_S2), W2.reshape(48, 16)).reshape(1024, 512).astype(bf)
    G3 = jnp.dot(jnp.asarray(_S3), W3.reshape(48, 16)).reshape(512, 256).astype(bf)
    return (G1, jnp.tile(b1, 64).reshape(1, 1024),
            G2, jnp.tile(b2, 32).reshape(1, 512),
            G3, jnp.tile(b3, 16).reshape(1, 256),
            ln_s.reshape(1, 256), ln_b.reshape(1, 256))


def _lidar_tc(fwd_raw, rear_raw, fws, rws):
    n = fwd_raw.shape[0]
    grid = n // _BB
    ws = list(fws) + list(rws)
    wspecs = [pl.BlockSpec(w.shape, functools.partial(
        lambda nd, i: (0,) * nd, w.ndim)) for w in ws]
    return pl.pallas_call(
        _lidar_body,
        grid=(grid,),
        in_specs=[pl.BlockSpec((_BB, 1024), lambda i: (i, 0)),
                  pl.BlockSpec((_BB, 1024), lambda i: (i, 0))] + wspecs,
        out_specs=pl.BlockSpec((_BB, 512), lambda i: (i, 0)),
        out_shape=jax.ShapeDtypeStruct((n, 512), jnp.float32),
        compiler_params=pltpu.CompilerParams(
            dimension_semantics=("arbitrary",)),
    )(fwd_raw, rear_raw, *ws)


def kernel(self_pos, teammate_positions, opponent_positions,
           opponent_last_known_positions, self_feat, fwd_lidar, rear_lidar,
           teammates, opponents, opponents_last_known, opponent_masks,
           agent_map, unmasked_agent_map, table, fW1, fb1, fW2, fb2, fW3, fb3,
           fln_s, fln_b, rW1, rb1, rW2, rb2, rW3, rb3, rln_s, rln_b, train):
    B = self_pos.shape[0]
    pos_all = jnp.concatenate([
        self_pos.reshape(-1, 2),
        teammate_positions.reshape(-1, 2),
        opponent_positions.reshape(-1, 2),
        opponent_last_known_positions.reshape(-1, 2),
    ], axis=0)
    px = pos_all[:, 0]
    py = pos_all[:, 1]
    t0 = table[:, :, 0]
    t1 = table[:, :, 1]
    o0, o1 = _sc_hash_encode(t0, t1, px, py)
    Q = px.shape[0]
    enc = jnp.stack([o0, o1], axis=-1)
    enc = enc.transpose(1, 0, 2).reshape(Q, 32)
    enc_self = enc[:B]
    enc_tm = enc[B:4 * B].reshape(B, 3, 32)
    enc_op = enc[4 * B:8 * B].reshape(B, 4, 32)
    enc_ol = enc[8 * B:12 * B].reshape(B, 4, 32)
    fws = _prep_w(fW1, fb1, fW2, fb2, fW3, fb3, fln_s, fln_b)
    rws = _prep_w(rW1, rb1, rW2, rb2, rW3, rb3, rln_s, rln_b)
    lid = _lidar_tc(fwd_lidar.reshape(B, 1024), rear_lidar.reshape(B, 1024),
                    fws, rws)
    self_ob = jnp.concatenate([enc_self, self_feat, lid], axis=-1)
    tm = jnp.concatenate([enc_tm, teammates], axis=-1)
    op = jnp.concatenate([enc_op, opponents], axis=-1)
    ol = jnp.concatenate([enc_ol, opponents_last_known], axis=-1)
    return (self_ob, tm, op, ol, opponent_masks, agent_map, unmasked_agent_map)
